# Initial kernel scaffold; baseline (speedup 1.0000x reference)
import jax
import jax.numpy as jnp
from jax.experimental import pallas as pl


def kernel(x, edge_index, bn0, enc, dec):
    raise NotImplementedError("write your pallas kernel here")


# speedup vs baseline: 1.6497x; 1.6497x over previous
import functools

import jax
import jax.numpy as jnp
from jax import lax
from jax.experimental import pallas as pl
from jax.experimental.pallas import tpu as pltpu
from jax.experimental.pallas import tpu_sc as plsc

_EPS = 1e-5
_BN = 2000
_BE = 1600
_C = 80


def _colstats_body(x_ref, s1_ref, s2_ref):
    i = pl.program_id(0)
    x = x_ref[...]
    d = x.shape[-1]

    @pl.when(i == 0)
    def _():
        s1_ref[...] = jnp.zeros_like(s1_ref)
        s2_ref[...] = jnp.zeros_like(s2_ref)

    s1_ref[...] += jnp.sum(x.reshape(-1, 8, d), axis=0)
    s2_ref[...] += jnp.sum((x * x).reshape(-1, 8, d), axis=0)


def _colstats(x):
    n, d = x.shape
    s1, s2 = pl.pallas_call(
        _colstats_body,
        grid=(n // _BN,),
        in_specs=[pl.BlockSpec((_BN, d), lambda i: (i, 0))],
        out_specs=(pl.BlockSpec((8, d), lambda i: (0, 0)),
                   pl.BlockSpec((8, d), lambda i: (0, 0))),
        out_shape=(jax.ShapeDtypeStruct((8, d), jnp.float32),
                   jax.ShapeDtypeStruct((8, d), jnp.float32)),
    )(x)
    return jnp.sum(s1, axis=0), jnp.sum(s2, axis=0)


def _mm_stats_body(relu, x_ref, wt_ref, b_ref, h_ref, s1_ref, s2_ref):
    i = pl.program_id(0)
    z = jnp.dot(x_ref[...], wt_ref[...],
                preferred_element_type=jnp.float32,
                precision=lax.Precision.HIGHEST) + b_ref[...]
    if relu:
        z = jnp.maximum(z, 0.0)
    h_ref[...] = z
    d = z.shape[-1]

    @pl.when(i == 0)
    def _():
        s1_ref[...] = jnp.zeros_like(s1_ref)
        s2_ref[...] = jnp.zeros_like(s2_ref)

    s1_ref[...] += jnp.sum(z.reshape(-1, 8, d), axis=0)
    s2_ref[...] += jnp.sum((z * z).reshape(-1, 8, d), axis=0)


def _mm_stats(x, wt, b, relu=True):
    e, din = x.shape
    dout = wt.shape[1]
    h, s1, s2 = pl.pallas_call(
        functools.partial(_mm_stats_body, relu),
        grid=(e // _BE,),
        in_specs=[pl.BlockSpec((_BE, din), lambda i: (i, 0)),
                  pl.BlockSpec((din, dout), lambda i: (0, 0)),
                  pl.BlockSpec((1, dout), lambda i: (0, 0))],
        out_specs=(pl.BlockSpec((_BE, dout), lambda i: (i, 0)),
                   pl.BlockSpec((8, dout), lambda i: (0, 0)),
                   pl.BlockSpec((8, dout), lambda i: (0, 0))),
        out_shape=(jax.ShapeDtypeStruct((e, dout), jnp.float32),
                   jax.ShapeDtypeStruct((8, dout), jnp.float32),
                   jax.ShapeDtypeStruct((8, dout), jnp.float32)),
    )(x, wt, b)
    return h, jnp.sum(s1, axis=0), jnp.sum(s2, axis=0)


def _mm_body(x_ref, wt_ref, b_ref, h_ref):
    h_ref[...] = jnp.dot(x_ref[...], wt_ref[...],
                         preferred_element_type=jnp.float32,
                precision=lax.Precision.HIGHEST) + b_ref[...]


def _mm_plain(x, wt, b):
    e, din = x.shape
    dout = wt.shape[1]
    return pl.pallas_call(
        _mm_body,
        grid=(e // _BE,),
        in_specs=[pl.BlockSpec((_BE, din), lambda i: (i, 0)),
                  pl.BlockSpec((din, dout), lambda i: (0, 0)),
                  pl.BlockSpec((1, dout), lambda i: (0, 0))],
        out_specs=pl.BlockSpec((_BE, dout), lambda i: (i, 0)),
        out_shape=jax.ShapeDtypeStruct((e, dout), jnp.float32),
    )(x, wt, b)


def _pq_body(x_ref, wt_ref, b_ref, p_ref, q_ref):
    z = jnp.dot(x_ref[...], wt_ref[...],
                preferred_element_type=jnp.float32,
                precision=lax.Precision.HIGHEST) + b_ref[...]
    h = z.shape[-1] // 2
    p_ref[...] = z[:, :h]
    q_ref[...] = z[:, h:]


def _pq(x, wt, b):
    n, din = x.shape
    dout = wt.shape[1] // 2
    return pl.pallas_call(
        _pq_body,
        grid=(n // _BN,),
        in_specs=[pl.BlockSpec((_BN, din), lambda i: (i, 0)),
                  pl.BlockSpec((din, 2 * dout), lambda i: (0, 0)),
                  pl.BlockSpec((1, 2 * dout), lambda i: (0, 0))],
        out_specs=(pl.BlockSpec((_BN, dout), lambda i: (i, 0)),
                   pl.BlockSpec((_BN, dout), lambda i: (i, 0))),
        out_shape=(jax.ShapeDtypeStruct((n, dout), jnp.float32),
                   jax.ShapeDtypeStruct((n, dout), jnp.float32)),
    )(x, wt, b)


_FB = 1000


def _fin_pq_body(hid, a_ref, s_ref, t_ref, wt_ref, b_ref, p_ref, q_ref):
    a = a_ref[0]
    cnt = a[:, hid:hid + 1]
    mean = a[:, :hid] / jnp.maximum(cnt, 1.0)
    g = jnp.where(cnt > 0, mean * s_ref[...] + t_ref[...], 0.0)
    z = jnp.dot(g, wt_ref[...], preferred_element_type=jnp.float32,
                precision=lax.Precision.HIGHEST) + b_ref[...]
    h = z.shape[-1] // 2
    p_ref[...] = z[:, :h]
    q_ref[...] = z[:, h:]


def _fin_pq(n, hid, a, s, t, wt, b):
    fpad = a.shape[-1]
    dout = wt.shape[1] // 2
    bh = (n // 2) // _FB
    return pl.pallas_call(
        functools.partial(_fin_pq_body, hid),
        grid=(n // _FB,),
        in_specs=[pl.BlockSpec((1, _FB, fpad), lambda i: (i // bh, i % bh, 0)),
                  pl.BlockSpec((1, hid), lambda i: (0, 0)),
                  pl.BlockSpec((1, hid), lambda i: (0, 0)),
                  pl.BlockSpec((hid, 2 * dout), lambda i: (0, 0)),
                  pl.BlockSpec((1, 2 * dout), lambda i: (0, 0))],
        out_specs=(pl.BlockSpec((_FB, dout), lambda i: (i, 0)),
                   pl.BlockSpec((_FB, dout), lambda i: (i, 0))),
        out_shape=(jax.ShapeDtypeStruct((n, dout), jnp.float32),
                   jax.ShapeDtypeStruct((n, dout), jnp.float32)),
    )(a, s, t, wt, b)


def _fin_out_body(hid, a_ref, ca_ref, o_ref):
    cnt = ca_ref[0][:, hid:hid + 1]
    m = a_ref[0] / jnp.maximum(cnt, 1.0)
    o_ref[...] = jnp.where(cnt > 0, m, 0.0)


def _fin_out(n, hid, a, ca):
    d = a.shape[-1]
    fpad = ca.shape[-1]
    bh = (n // 2) // _FB
    return pl.pallas_call(
        functools.partial(_fin_out_body, hid),
        grid=(n // _FB,),
        in_specs=[pl.BlockSpec((1, _FB, d), lambda i: (i // bh, i % bh, 0)),
                  pl.BlockSpec((1, _FB, fpad), lambda i: (i // bh, i % bh, 0))],
        out_specs=pl.BlockSpec((_FB, d), lambda i: (i, 0)),
        out_shape=jax.ShapeDtypeStruct((n, d), jnp.float32),
    )(a, ca)


def _sc_mesh():
    return plsc.VectorSubcoreMesh(core_axis_name="c", subcore_axis_name="s")


@functools.cache
def _make_edge_pass(e, n, d):
    nw = 32
    et = e // nw
    nchunk = et // _C
    ng = d // 16

    @functools.partial(
        pl.kernel,
        mesh=_sc_mesh(),
        out_type=(jax.ShapeDtypeStruct((e, d), jnp.float32),
                  jax.ShapeDtypeStruct((nw, 1, 2 * d), jnp.float32)),
        scratch_types=[
            pltpu.VMEM((nchunk, _C), jnp.int32),
            pltpu.VMEM((nchunk, _C), jnp.int32),
            pltpu.VMEM((_C, d), jnp.float32),
            pltpu.VMEM((_C, d), jnp.float32),
            pltpu.VMEM((1, 2 * d), jnp.float32),
            pltpu.SemaphoreType.DMA,
            pltpu.SemaphoreType.DMA,
        ],
    )
    def edge_pass(p_hbm, q_hbm, src3_hbm, dst3_hbm, h_hbm, st_hbm,
                  idxs, idxd, bufp, bufq, acc, semp, semq):
        c = lax.axis_index("c")
        s = lax.axis_index("s")
        wid = s * 2 + c
        zero16 = jnp.zeros((16,), jnp.float32)
        for j in range(2 * ng):
            acc[0, pl.ds(j * 16, 16)] = zero16
        pltpu.sync_copy(src3_hbm.at[wid], idxs)
        pltpu.sync_copy(dst3_hbm.at[wid], idxd)
        row_base = wid * nchunk

        def chunk_body(j, carry):
            cp = pltpu.async_copy(p_hbm.at[idxd.at[j]], bufp, semp)
            cq = pltpu.async_copy(q_hbm.at[idxs.at[j]], bufq, semq)
            cp.wait()
            cq.wait()
            for k in range(ng):
                col = k * 16

                def rows(r8, ca):
                    s1, s2 = ca
                    for u in range(8):
                        r = r8 * 8 + u
                        hv = jnp.maximum(
                            bufp[r, pl.ds(col, 16)]
                            + bufq[r, pl.ds(col, 16)], 0.0)
                        bufp[r, pl.ds(col, 16)] = hv
                        s1 = s1 + hv
                        s2 = s2 + hv * hv
                    return (s1, s2)

                s1, s2 = lax.fori_loop(0, _C // 8, rows, (zero16, zero16))
                acc[0, pl.ds(col, 16)] += s1
                acc[0, pl.ds(d + col, 16)] += s2
            pltpu.sync_copy(bufp, h_hbm.at[pl.ds((row_base + j) * _C, _C)])
            return carry

        lax.fori_loop(0, nchunk, chunk_body, 0)
        pltpu.sync_copy(acc, st_hbm.at[wid])

    return edge_pass


@functools.cache
def _make_scatter(e, n, f):
    nh = n // 2
    nloc = nh + 8
    et = e // 16
    nchunk = et // _C
    nr = (nh // 16) // 8 * 8
    nz = nr // 3
    rem = nloc - 16 * nr

    @functools.partial(
        pl.kernel,
        mesh=_sc_mesh(),
        out_type=jax.ShapeDtypeStruct((2, nloc, f), jnp.float32),
        scratch_types=[
            pltpu.VMEM((nchunk, _C), jnp.int32),
            pltpu.VMEM((_C, f), jnp.float32),
            pltpu.VMEM((nz, f), jnp.float32),
            pltpu.VMEM_SHARED((nloc, f), jnp.float32),
        ],
    )
    def scatter(h_hbm, dst16_hbm, out_hbm, idx, buf, zbuf, shacc):
        c = lax.axis_index("c")
        s = lax.axis_index("s")
        lo = c * nh
        zero16 = jnp.zeros((16,), jnp.float32)

        def zrow(r, carry):
            for k in range(f // 16):
                zbuf[r, pl.ds(k * 16, 16)] = zero16
            return carry

        lax.fori_loop(0, nz, zrow, 0)

        for j in range(3):
            pltpu.sync_copy(zbuf, shacc.at[pl.ds(s * nr + j * nz, nz)])

        @pl.when(s == 15)
        def _():
            pltpu.sync_copy(zbuf.at[pl.ds(0, rem)],
                            shacc.at[pl.ds(16 * nr, rem)])

        pltpu.sync_copy(dst16_hbm.at[s], idx)
        row_base = s * nchunk

        def remap_body(j, carry):
            for g in range(_C // 16):
                v = idx[j, pl.ds(g * 16, 16)]
                inb = (v >= lo) & (v < lo + nh)
                idx[j, pl.ds(g * 16, 16)] = jnp.where(inb, v - lo, nh)
            return carry

        lax.fori_loop(0, nchunk, remap_body, 0)
        plsc.subcore_barrier()

        def chunk_body(j, carry):
            pltpu.sync_copy(h_hbm.at[pl.ds((row_base + j) * _C, _C)], buf)
            pltpu.sync_copy(buf, shacc.at[idx.at[j]], add=True)
            return carry

        lax.fori_loop(0, nchunk, chunk_body, 0)
        plsc.subcore_barrier()
        pltpu.sync_copy(shacc.at[pl.ds(s * nr, nr)],
                        out_hbm.at[c, pl.ds(s * nr, nr)])

        @pl.when(s == 15)
        def _():
            pltpu.sync_copy(shacc.at[pl.ds(16 * nr, rem)],
                            out_hbm.at[c, pl.ds(16 * nr, rem)])

    return scatter


def _bn_affine(g, bt, s1, s2, count):
    mu = s1 / count
    var = s2 / count - mu * mu
    s = g * lax.rsqrt(var + _EPS)
    return s, bt - s * mu


def kernel(x, edge_index, bn0, enc, dec):
    n, d = x.shape
    e = edge_index.shape[1]
    src = edge_index[0]
    dst = edge_index[1]
    src3 = src.reshape(32, e // (32 * _C), _C)
    dst3 = dst.reshape(32, e // (32 * _C), _C)
    dst16 = dst.reshape(16, e // (16 * _C), _C)

    s1x, s2x = _colstats(x)
    s0, t0 = _bn_affine(bn0[0], bn0[1], s1x, s2x, n)

    W1, b1 = enc[0][0], enc[0][1]
    W1a, W1b = W1[:, :d], W1[:, d:]
    A1 = (W1a - W1b) * s0[None, :]
    B1 = W1b * s0[None, :]
    b1p = b1 + W1a @ t0
    big = W1.shape[0]
    wt1 = jnp.concatenate([A1, B1], axis=0).T
    bias1 = jnp.concatenate([b1p, jnp.zeros_like(b1p)])[None, :]
    p1, q1 = _pq(x, wt1, bias1)

    edge_pass = _make_edge_pass(e, n, big)
    h, st = edge_pass(p1, q1, src3, dst3)
    stsum = jnp.sum(st[:, 0, :], axis=0)
    s, t = _bn_affine(enc[0][2], enc[0][3], stsum[:big], stsum[big:], e)

    for layer in enc[1:-1]:
        W, b, g, bt = layer
        wt = (W * s[None, :]).T
        bp = (b + W @ t)[None, :]
        h, hs1, hs2 = _mm_stats(h, wt, bp, relu=True)
        s, t = _bn_affine(g, bt, hs1, hs2, e)

    W, b, g, bt = enc[-1]
    hid = W.shape[0]
    fpad = 128
    wt = jnp.zeros((big, fpad), jnp.float32).at[:, :hid].set((W * s[None, :]).T)
    bp = jnp.zeros((1, fpad), jnp.float32).at[:, :hid].set((b + W @ t)[None, :])
    bp = bp.at[0, hid].set(1.0)
    h, hs1, hs2 = _mm_stats(h, wt, bp, relu=True)
    s, t = _bn_affine(g, bt, hs1[:hid], hs2[:hid], e)

    scat = _make_scatter(e, n, fpad)
    sums = scat(h, dst16)

    V1, c1 = dec[0][0], dec[0][1]
    V1a, V1b = V1[:, :hid], V1[:, hid:]
    wt2 = jnp.concatenate([V1a - V1b, V1b], axis=0).T
    bias2 = jnp.concatenate([c1, jnp.zeros_like(c1)])[None, :]
    p2, q2 = _fin_pq(n, hid, sums, s[None, :], t[None, :], wt2, bias2)

    h, st = edge_pass(p2, q2, src3, dst3)
    stsum = jnp.sum(st[:, 0, :], axis=0)
    s, t = _bn_affine(dec[0][2], dec[0][3], stsum[:big], stsum[big:], e)

    for layer in dec[1:-1]:
        W, b, g, bt = layer
        wt = (W * s[None, :]).T
        bp = (b + W @ t)[None, :]
        h, hs1, hs2 = _mm_stats(h, wt, bp, relu=True)
        s, t = _bn_affine(g, bt, hs1, hs2, e)

    Wf, bf = dec[-1]
    wtf = (Wf * s[None, :]).T
    bfp = (bf + Wf @ t)[None, :]
    h = _mm_plain(h, wtf, bfp)

    sums2 = scat(h, dst16)
    return _fin_out(n, hid, sums2, sums)


# speedup vs baseline: 1.7597x; 1.0667x over previous
import functools

import jax
import jax.numpy as jnp
from jax import lax
from jax.experimental import pallas as pl
from jax.experimental.pallas import tpu as pltpu
from jax.experimental.pallas import tpu_sc as plsc

_EPS = 1e-5
_BN = 2000
_BE = 1600
_C = 80


def _colstats_body(x_ref, s1_ref, s2_ref):
    i = pl.program_id(0)
    x = x_ref[...]
    d = x.shape[-1]

    @pl.when(i == 0)
    def _():
        s1_ref[...] = jnp.zeros_like(s1_ref)
        s2_ref[...] = jnp.zeros_like(s2_ref)

    s1_ref[...] += jnp.sum(x.reshape(-1, 8, d), axis=0)
    s2_ref[...] += jnp.sum((x * x).reshape(-1, 8, d), axis=0)


def _colstats(x):
    n, d = x.shape
    s1, s2 = pl.pallas_call(
        _colstats_body,
        grid=(n // _BN,),
        in_specs=[pl.BlockSpec((_BN, d), lambda i: (i, 0))],
        out_specs=(pl.BlockSpec((8, d), lambda i: (0, 0)),
                   pl.BlockSpec((8, d), lambda i: (0, 0))),
        out_shape=(jax.ShapeDtypeStruct((8, d), jnp.float32),
                   jax.ShapeDtypeStruct((8, d), jnp.float32)),
    )(x)
    return jnp.sum(s1, axis=0), jnp.sum(s2, axis=0)


def _mm_stats_body(relu, out_bf16, x_ref, wt_ref, b_ref,
                   h_ref, s1_ref, s2_ref):
    i = pl.program_id(0)
    z = jnp.dot(x_ref[...], wt_ref[...],
                preferred_element_type=jnp.float32,
                precision=lax.Precision.HIGHEST) + b_ref[...]
    if relu:
        z = jnp.maximum(z, 0.0)
    if out_bf16:
        hc = z.astype(jnp.bfloat16)
        h_ref[...] = hc
        z = hc.astype(jnp.float32)
    else:
        h_ref[...] = z
    d = z.shape[-1]

    @pl.when(i == 0)
    def _():
        s1_ref[...] = jnp.zeros_like(s1_ref)
        s2_ref[...] = jnp.zeros_like(s2_ref)

    s1_ref[...] += jnp.sum(z.reshape(-1, 8, d), axis=0)
    s2_ref[...] += jnp.sum((z * z).reshape(-1, 8, d), axis=0)


def _mm_stats(x, wt, b, relu=True, out_bf16=False):
    e, din = x.shape
    dout = wt.shape[1]
    odt = jnp.bfloat16 if out_bf16 else jnp.float32
    h, s1, s2 = pl.pallas_call(
        functools.partial(_mm_stats_body, relu, out_bf16),
        grid=(e // _BE,),
        in_specs=[pl.BlockSpec((_BE, din), lambda i: (i, 0)),
                  pl.BlockSpec((din, dout), lambda i: (0, 0)),
                  pl.BlockSpec((1, dout), lambda i: (0, 0))],
        out_specs=(pl.BlockSpec((_BE, dout), lambda i: (i, 0)),
                   pl.BlockSpec((8, dout), lambda i: (0, 0)),
                   pl.BlockSpec((8, dout), lambda i: (0, 0))),
        out_shape=(jax.ShapeDtypeStruct((e, dout), odt),
                   jax.ShapeDtypeStruct((8, dout), jnp.float32),
                   jax.ShapeDtypeStruct((8, dout), jnp.float32)),
    )(x, wt, b)
    return h, jnp.sum(s1, axis=0), jnp.sum(s2, axis=0)


def _mm_body(x_ref, wt_ref, b_ref, h_ref):
    h_ref[...] = jnp.dot(x_ref[...], wt_ref[...],
                         preferred_element_type=jnp.float32,
                         precision=lax.Precision.HIGHEST) + b_ref[...]


def _mm_plain(x, wt, b):
    e, din = x.shape
    dout = wt.shape[1]
    return pl.pallas_call(
        _mm_body,
        grid=(e // _BE,),
        in_specs=[pl.BlockSpec((_BE, din), lambda i: (i, 0)),
                  pl.BlockSpec((din, dout), lambda i: (0, 0)),
                  pl.BlockSpec((1, dout), lambda i: (0, 0))],
        out_specs=pl.BlockSpec((_BE, dout), lambda i: (i, 0)),
        out_shape=jax.ShapeDtypeStruct((e, dout), jnp.float32),
    )(x, wt, b)


def _pq_body(x_ref, wt_ref, b_ref, p_ref, q_ref):
    z = jnp.dot(x_ref[...], wt_ref[...],
                preferred_element_type=jnp.float32,
                precision=lax.Precision.HIGHEST) + b_ref[...]
    h = z.shape[-1] // 2
    p_ref[...] = z[:, :h]
    q_ref[...] = z[:, h:]


def _pq(x, wt, b):
    n, din = x.shape
    dout = wt.shape[1] // 2
    return pl.pallas_call(
        _pq_body,
        grid=(n // _BN,),
        in_specs=[pl.BlockSpec((_BN, din), lambda i: (i, 0)),
                  pl.BlockSpec((din, 2 * dout), lambda i: (0, 0)),
                  pl.BlockSpec((1, 2 * dout), lambda i: (0, 0))],
        out_specs=(pl.BlockSpec((_BN, dout), lambda i: (i, 0)),
                   pl.BlockSpec((_BN, dout), lambda i: (i, 0))),
        out_shape=(jax.ShapeDtypeStruct((n, dout), jnp.float32),
                   jax.ShapeDtypeStruct((n, dout), jnp.float32)),
    )(x, wt, b)


_FB = 1000


def _fin_pq_body(hid, a_ref, s_ref, t_ref, wt_ref, b_ref, p_ref, q_ref):
    a = a_ref[0]
    cnt = a[:, hid:hid + 1]
    mean = a[:, :hid] / jnp.maximum(cnt, 1.0)
    g = jnp.where(cnt > 0, mean * s_ref[...] + t_ref[...], 0.0)
    z = jnp.dot(g, wt_ref[...], preferred_element_type=jnp.float32,
                precision=lax.Precision.HIGHEST) + b_ref[...]
    h = z.shape[-1] // 2
    p_ref[...] = z[:, :h]
    q_ref[...] = z[:, h:]


def _fin_pq(n, hid, a, s, t, wt, b):
    fpad = a.shape[-1]
    dout = wt.shape[1] // 2
    bh = (n // 2) // _FB
    return pl.pallas_call(
        functools.partial(_fin_pq_body, hid),
        grid=(n // _FB,),
        in_specs=[pl.BlockSpec((1, _FB, fpad), lambda i: (i // bh, i % bh, 0)),
                  pl.BlockSpec((1, hid), lambda i: (0, 0)),
                  pl.BlockSpec((1, hid), lambda i: (0, 0)),
                  pl.BlockSpec((hid, 2 * dout), lambda i: (0, 0)),
                  pl.BlockSpec((1, 2 * dout), lambda i: (0, 0))],
        out_specs=(pl.BlockSpec((_FB, dout), lambda i: (i, 0)),
                   pl.BlockSpec((_FB, dout), lambda i: (i, 0))),
        out_shape=(jax.ShapeDtypeStruct((n, dout), jnp.float32),
                   jax.ShapeDtypeStruct((n, dout), jnp.float32)),
    )(a, s, t, wt, b)


def _fin_out_body(hid, a_ref, ca_ref, o_ref):
    cnt = ca_ref[0][:, hid:hid + 1]
    m = a_ref[0] / jnp.maximum(cnt, 1.0)
    o_ref[...] = jnp.where(cnt > 0, m, 0.0)


def _fin_out(n, hid, a, ca):
    d = a.shape[-1]
    fpad = ca.shape[-1]
    bh = (n // 2) // _FB
    return pl.pallas_call(
        functools.partial(_fin_out_body, hid),
        grid=(n // _FB,),
        in_specs=[pl.BlockSpec((1, _FB, d), lambda i: (i // bh, i % bh, 0)),
                  pl.BlockSpec((1, _FB, fpad), lambda i: (i // bh, i % bh, 0))],
        out_specs=pl.BlockSpec((_FB, d), lambda i: (i, 0)),
        out_shape=jax.ShapeDtypeStruct((n, d), jnp.float32),
    )(a, ca)


def _sc_mesh():
    return plsc.VectorSubcoreMesh(core_axis_name="c", subcore_axis_name="s")


@functools.cache
def _make_edge_pass(e, n, d):
    nw = 32
    et = e // nw
    nchunk = et // _C
    ng = d // 16

    @functools.partial(
        pl.kernel,
        mesh=_sc_mesh(),
        out_type=(jax.ShapeDtypeStruct((e, d), jnp.float32),
                  jax.ShapeDtypeStruct((nw, 1, 2 * d), jnp.float32)),
        scratch_types=[
            pltpu.VMEM((nchunk, _C), jnp.int32),
            pltpu.VMEM((nchunk, _C), jnp.int32),
            pltpu.VMEM((_C, d), jnp.float32),
            pltpu.VMEM((_C, d), jnp.float32),
            pltpu.VMEM((_C, d), jnp.float32),
            pltpu.VMEM((_C, d), jnp.float32),
            pltpu.VMEM((1, 2 * d), jnp.float32),
            pltpu.SemaphoreType.DMA,
            pltpu.SemaphoreType.DMA,
            pltpu.SemaphoreType.DMA,
            pltpu.SemaphoreType.DMA,
            pltpu.SemaphoreType.DMA,
        ],
    )
    def edge_pass(p_hbm, q_hbm, src3_hbm, dst3_hbm, h_hbm, st_hbm,
                  idxs, idxd, bufp0, bufq0, bufp1, bufq1, acc,
                  sp0, sq0, sp1, sq1, wbs):
        c = lax.axis_index("c")
        s = lax.axis_index("s")
        wid = s * 2 + c
        zero16 = jnp.zeros((16,), jnp.float32)
        for j in range(2 * ng):
            acc[0, pl.ds(j * 16, 16)] = zero16
        pltpu.sync_copy(src3_hbm.at[wid], idxs)
        pltpu.sync_copy(dst3_hbm.at[wid], idxd)
        row_base = wid * nchunk

        def compute(bufp, bufq):
            for k in range(ng):
                col = k * 16

                def rows(r8, ca):
                    s1, s2 = ca
                    for u in range(8):
                        r = r8 * 8 + u
                        hv = jnp.maximum(
                            bufp[r, pl.ds(col, 16)]
                            + bufq[r, pl.ds(col, 16)], 0.0)
                        bufp[r, pl.ds(col, 16)] = hv
                        s1 = s1 + hv
                        s2 = s2 + hv * hv
                    return (s1, s2)

                s1, s2 = lax.fori_loop(0, _C // 8, rows, (zero16, zero16))
                acc[0, pl.ds(col, 16)] += s1
                acc[0, pl.ds(d + col, 16)] += s2

        def body(i2, carry):
            j0 = i2 * 2
            j1 = j0 + 1
            g0p = pltpu.async_copy(p_hbm.at[idxd.at[j0]], bufp0, sp0)
            g0q = pltpu.async_copy(q_hbm.at[idxs.at[j0]], bufq0, sq0)
            g1p = pltpu.async_copy(p_hbm.at[idxd.at[j1]], bufp1, sp1)
            g1q = pltpu.async_copy(q_hbm.at[idxs.at[j1]], bufq1, sq1)
            g0p.wait()
            g0q.wait()
            compute(bufp0, bufq0)
            wb0 = pltpu.async_copy(
                bufp0, h_hbm.at[pl.ds((row_base + j0) * _C, _C)], wbs)
            g1p.wait()
            g1q.wait()
            compute(bufp1, bufq1)
            wb1 = pltpu.async_copy(
                bufp1, h_hbm.at[pl.ds((row_base + j1) * _C, _C)], wbs)
            wb0.wait()
            wb1.wait()
            return carry

        lax.fori_loop(0, nchunk // 2, body, 0)
        if nchunk % 2:
            j = nchunk - 1
            cp = pltpu.async_copy(p_hbm.at[idxd.at[j]], bufp0, sp0)
            cq = pltpu.async_copy(q_hbm.at[idxs.at[j]], bufq0, sq0)
            cp.wait()
            cq.wait()
            compute(bufp0, bufq0)
            pltpu.sync_copy(bufp0, h_hbm.at[pl.ds((row_base + j) * _C, _C)])
        pltpu.sync_copy(acc, st_hbm.at[wid])

    return edge_pass


@functools.cache
def _make_scatter(e, n, f):
    nh = n // 2
    nloc = nh + 8
    et = e // 16
    nchunk = et // _C
    nr = (nh // 16) // 8 * 8
    nz = nr // 3
    rem = nloc - 16 * nr

    @functools.partial(
        pl.kernel,
        mesh=_sc_mesh(),
        out_type=jax.ShapeDtypeStruct((2, nloc, f), jnp.float32),
        scratch_types=[
            pltpu.VMEM((nchunk, _C), jnp.int32),
            pltpu.VMEM((_C, f), jnp.float32),
            pltpu.VMEM((_C, f), jnp.float32),
            pltpu.VMEM((nz, f), jnp.float32),
            pltpu.VMEM_SHARED((nloc, f), jnp.float32),
            pltpu.SemaphoreType.DMA,
            pltpu.SemaphoreType.DMA,
            pltpu.SemaphoreType.DMA,
        ],
    )
    def scatter(h_hbm, dst16_hbm, out_hbm, idx, buf0, buf1, zbuf, shacc,
                sl0, sl1, sadd):
        c = lax.axis_index("c")
        s = lax.axis_index("s")
        lo = c * nh
        zero16 = jnp.zeros((16,), jnp.float32)

        def zrow(r, carry):
            for k in range(f // 16):
                zbuf[r, pl.ds(k * 16, 16)] = zero16
            return carry

        lax.fori_loop(0, nz, zrow, 0)

        for j in range(3):
            pltpu.sync_copy(zbuf, shacc.at[pl.ds(s * nr + j * nz, nz)])

        @pl.when(s == 15)
        def _():
            pltpu.sync_copy(zbuf.at[pl.ds(0, rem)],
                            shacc.at[pl.ds(16 * nr, rem)])

        pltpu.sync_copy(dst16_hbm.at[s], idx)
        row_base = s * nchunk

        def remap_body(j, carry):
            for g in range(_C // 16):
                v = idx[j, pl.ds(g * 16, 16)]
                inb = (v >= lo) & (v < lo + nh)
                idx[j, pl.ds(g * 16, 16)] = jnp.where(inb, v - lo, nh)
            return carry

        lax.fori_loop(0, nchunk, remap_body, 0)
        plsc.subcore_barrier()

        def chunk_body(i2, carry):
            j0 = i2 * 2
            j1 = j0 + 1
            l0 = pltpu.async_copy(
                h_hbm.at[pl.ds((row_base + j0) * _C, _C)], buf0, sl0)
            l1 = pltpu.async_copy(
                h_hbm.at[pl.ds((row_base + j1) * _C, _C)], buf1, sl1)
            l0.wait()
            a0 = pltpu.async_copy(buf0, shacc.at[idx.at[j0]], sadd, add=True)
            l1.wait()
            a1 = pltpu.async_copy(buf1, shacc.at[idx.at[j1]], sadd, add=True)
            a0.wait()
            a1.wait()
            return carry

        lax.fori_loop(0, nchunk // 2, chunk_body, 0)
        plsc.subcore_barrier()
        pltpu.sync_copy(shacc.at[pl.ds(s * nr, nr)],
                        out_hbm.at[c, pl.ds(s * nr, nr)])

        @pl.when(s == 15)
        def _():
            pltpu.sync_copy(shacc.at[pl.ds(16 * nr, rem)],
                            out_hbm.at[c, pl.ds(16 * nr, rem)])

    return scatter


def _bn_affine(g, bt, s1, s2, count):
    mu = s1 / count
    var = s2 / count - mu * mu
    s = g * lax.rsqrt(var + _EPS)
    return s, bt - s * mu


def kernel(x, edge_index, bn0, enc, dec):
    n, d = x.shape
    e = edge_index.shape[1]
    src = edge_index[0]
    dst = edge_index[1]
    src3 = src.reshape(32, e // (32 * _C), _C)
    dst3 = dst.reshape(32, e // (32 * _C), _C)
    dst16 = dst.reshape(16, e // (16 * _C), _C)

    s1x, s2x = _colstats(x)
    s0, t0 = _bn_affine(bn0[0], bn0[1], s1x, s2x, n)

    W1, b1 = enc[0][0], enc[0][1]
    W1a, W1b = W1[:, :d], W1[:, d:]
    A1 = (W1a - W1b) * s0[None, :]
    B1 = W1b * s0[None, :]
    b1p = b1 + W1a @ t0
    big = W1.shape[0]
    wt1 = jnp.concatenate([A1, B1], axis=0).T
    bias1 = jnp.concatenate([b1p, jnp.zeros_like(b1p)])[None, :]
    p1, q1 = _pq(x, wt1, bias1)

    edge_pass = _make_edge_pass(e, n, big)
    h, st = edge_pass(p1, q1, src3, dst3)
    stsum = jnp.sum(st[:, 0, :], axis=0)
    s, t = _bn_affine(enc[0][2], enc[0][3], stsum[:big], stsum[big:], e)

    for layer in enc[1:-1]:
        W, b, g, bt = layer
        wt = (W * s[None, :]).T
        bp = (b + W @ t)[None, :]
        h, hs1, hs2 = _mm_stats(h, wt, bp, relu=True)
        s, t = _bn_affine(g, bt, hs1, hs2, e)

    W, b, g, bt = enc[-1]
    hid = W.shape[0]
    fpad = 128
    wt = jnp.zeros((big, fpad), jnp.float32).at[:, :hid].set((W * s[None, :]).T)
    bp = jnp.zeros((1, fpad), jnp.float32).at[:, :hid].set((b + W @ t)[None, :])
    bp = bp.at[0, hid].set(1.0)
    h, hs1, hs2 = _mm_stats(h, wt, bp, relu=True)
    s, t = _bn_affine(g, bt, hs1[:hid], hs2[:hid], e)

    scat = _make_scatter(e, n, fpad)
    sums = scat(h, dst16)

    V1, c1 = dec[0][0], dec[0][1]
    V1a, V1b = V1[:, :hid], V1[:, hid:]
    wt2 = jnp.concatenate([V1a - V1b, V1b], axis=0).T
    bias2 = jnp.concatenate([c1, jnp.zeros_like(c1)])[None, :]
    p2, q2 = _fin_pq(n, hid, sums, s[None, :], t[None, :], wt2, bias2)

    h, st = edge_pass(p2, q2, src3, dst3)
    stsum = jnp.sum(st[:, 0, :], axis=0)
    s, t = _bn_affine(dec[0][2], dec[0][3], stsum[:big], stsum[big:], e)

    for layer in dec[1:-1]:
        W, b, g, bt = layer
        wt = (W * s[None, :]).T
        bp = (b + W @ t)[None, :]
        h, hs1, hs2 = _mm_stats(h, wt, bp, relu=True)
        s, t = _bn_affine(g, bt, hs1, hs2, e)

    Wf, bf = dec[-1]
    wtf = (Wf * s[None, :]).T
    bfp = (bf + Wf @ t)[None, :]
    h = _mm_plain(h, wtf, bfp)

    sums2 = scat(h, dst16)
    return _fin_out(n, hid, sums2, sums)


# speedup vs baseline: 1.9591x; 1.1133x over previous
import functools

import jax
import jax.numpy as jnp
from jax import lax
from jax.experimental import pallas as pl
from jax.experimental.pallas import tpu as pltpu
from jax.experimental.pallas import tpu_sc as plsc

_EPS = 1e-5
_BN = 2000
_BE = 3200
_C = 80


def _colstats_body(x_ref, s1_ref, s2_ref):
    i = pl.program_id(0)
    x = x_ref[...]
    d = x.shape[-1]

    @pl.when(i == 0)
    def _():
        s1_ref[...] = jnp.zeros_like(s1_ref)
        s2_ref[...] = jnp.zeros_like(s2_ref)

    s1_ref[...] += jnp.sum(x.reshape(-1, 8, d), axis=0)
    s2_ref[...] += jnp.sum((x * x).reshape(-1, 8, d), axis=0)


def _colstats(x):
    n, d = x.shape
    s1, s2 = pl.pallas_call(
        _colstats_body,
        grid=(n // _BN,),
        in_specs=[pl.BlockSpec((_BN, d), lambda i: (i, 0))],
        out_specs=(pl.BlockSpec((8, d), lambda i: (0, 0)),
                   pl.BlockSpec((8, d), lambda i: (0, 0))),
        out_shape=(jax.ShapeDtypeStruct((8, d), jnp.float32),
                   jax.ShapeDtypeStruct((8, d), jnp.float32)),
    )(x)
    return jnp.sum(s1, axis=0), jnp.sum(s2, axis=0)


def _mm_stats_body(relu, out_bf16, x_ref, wt_ref, b_ref,
                   h_ref, s1_ref, s2_ref):
    i = pl.program_id(0)
    z = jnp.dot(x_ref[...], wt_ref[...],
                preferred_element_type=jnp.float32,
                precision=lax.Precision.HIGHEST) + b_ref[...]
    if relu:
        z = jnp.maximum(z, 0.0)
    if out_bf16:
        hc = z.astype(jnp.bfloat16)
        h_ref[...] = hc
        z = hc.astype(jnp.float32)
    else:
        h_ref[...] = z
    d = z.shape[-1]

    @pl.when(i == 0)
    def _():
        s1_ref[...] = jnp.zeros_like(s1_ref)
        s2_ref[...] = jnp.zeros_like(s2_ref)

    s1_ref[...] += jnp.sum(z.reshape(-1, 8, d), axis=0)
    s2_ref[...] += jnp.sum((z * z).reshape(-1, 8, d), axis=0)


def _mm_stats(x, wt, b, relu=True, out_bf16=False):
    e, din = x.shape
    dout = wt.shape[1]
    odt = jnp.bfloat16 if out_bf16 else jnp.float32
    h, s1, s2 = pl.pallas_call(
        functools.partial(_mm_stats_body, relu, out_bf16),
        grid=(e // _BE,),
        in_specs=[pl.BlockSpec((_BE, din), lambda i: (i, 0)),
                  pl.BlockSpec((din, dout), lambda i: (0, 0)),
                  pl.BlockSpec((1, dout), lambda i: (0, 0))],
        out_specs=(pl.BlockSpec((_BE, dout), lambda i: (i, 0)),
                   pl.BlockSpec((8, dout), lambda i: (0, 0)),
                   pl.BlockSpec((8, dout), lambda i: (0, 0))),
        out_shape=(jax.ShapeDtypeStruct((e, dout), odt),
                   jax.ShapeDtypeStruct((8, dout), jnp.float32),
                   jax.ShapeDtypeStruct((8, dout), jnp.float32)),
    )(x, wt, b)
    return h, jnp.sum(s1, axis=0), jnp.sum(s2, axis=0)


def _mm_body(x_ref, wt_ref, b_ref, h_ref):
    xb = x_ref[...].astype(jnp.bfloat16)
    h_ref[...] = jnp.dot(xb, wt_ref[...].astype(jnp.bfloat16),
                         preferred_element_type=jnp.float32) + b_ref[...]


def _mm_plain(x, wt, b):
    e, din = x.shape
    dout = wt.shape[1]
    return pl.pallas_call(
        _mm_body,
        grid=(e // _BE,),
        in_specs=[pl.BlockSpec((_BE, din), lambda i: (i, 0)),
                  pl.BlockSpec((din, dout), lambda i: (0, 0)),
                  pl.BlockSpec((1, dout), lambda i: (0, 0))],
        out_specs=pl.BlockSpec((_BE, dout), lambda i: (i, 0)),
        out_shape=jax.ShapeDtypeStruct((e, dout), jnp.float32),
    )(x, wt, b)


def _pq_body(x_ref, wt_ref, b_ref, p_ref, q_ref):
    z = jnp.dot(x_ref[...], wt_ref[...],
                preferred_element_type=jnp.float32,
                precision=lax.Precision.HIGHEST) + b_ref[...]
    h = z.shape[-1] // 2
    p_ref[...] = z[:, :h]
    q_ref[...] = z[:, h:]


def _pq(x, wt, b):
    n, din = x.shape
    dout = wt.shape[1] // 2
    return pl.pallas_call(
        _pq_body,
        grid=(n // _BN,),
        in_specs=[pl.BlockSpec((_BN, din), lambda i: (i, 0)),
                  pl.BlockSpec((din, 2 * dout), lambda i: (0, 0)),
                  pl.BlockSpec((1, 2 * dout), lambda i: (0, 0))],
        out_specs=(pl.BlockSpec((_BN, dout), lambda i: (i, 0)),
                   pl.BlockSpec((_BN, dout), lambda i: (i, 0))),
        out_shape=(jax.ShapeDtypeStruct((n, dout), jnp.float32),
                   jax.ShapeDtypeStruct((n, dout), jnp.float32)),
    )(x, wt, b)


_FB = 1000


def _fin_pq_body(hid, a_ref, s_ref, t_ref, wt_ref, b_ref, p_ref, q_ref):
    a = a_ref[0]
    cnt = a[:, hid:hid + 1]
    mean = a[:, :hid] / jnp.maximum(cnt, 1.0)
    g = jnp.where(cnt > 0, mean * s_ref[...] + t_ref[...], 0.0)
    z = jnp.dot(g, wt_ref[...], preferred_element_type=jnp.float32,
                precision=lax.Precision.HIGHEST) + b_ref[...]
    h = z.shape[-1] // 2
    p_ref[...] = z[:, :h]
    q_ref[...] = z[:, h:]


def _fin_pq(n, hid, a, s, t, wt, b):
    fpad = a.shape[-1]
    dout = wt.shape[1] // 2
    bh = (n // 2) // _FB
    return pl.pallas_call(
        functools.partial(_fin_pq_body, hid),
        grid=(n // _FB,),
        in_specs=[pl.BlockSpec((1, _FB, fpad), lambda i: (i // bh, i % bh, 0)),
                  pl.BlockSpec((1, hid), lambda i: (0, 0)),
                  pl.BlockSpec((1, hid), lambda i: (0, 0)),
                  pl.BlockSpec((hid, 2 * dout), lambda i: (0, 0)),
                  pl.BlockSpec((1, 2 * dout), lambda i: (0, 0))],
        out_specs=(pl.BlockSpec((_FB, dout), lambda i: (i, 0)),
                   pl.BlockSpec((_FB, dout), lambda i: (i, 0))),
        out_shape=(jax.ShapeDtypeStruct((n, dout), jnp.float32),
                   jax.ShapeDtypeStruct((n, dout), jnp.float32)),
    )(a, s, t, wt, b)


def _fin_out_body(hid, a_ref, ca_ref, o_ref):
    cnt = ca_ref[0][:, hid:hid + 1]
    m = a_ref[0] / jnp.maximum(cnt, 1.0)
    o_ref[...] = jnp.where(cnt > 0, m, 0.0)


def _fin_out(n, hid, a, ca):
    d = a.shape[-1]
    fpad = ca.shape[-1]
    bh = (n // 2) // _FB
    return pl.pallas_call(
        functools.partial(_fin_out_body, hid),
        grid=(n // _FB,),
        in_specs=[pl.BlockSpec((1, _FB, d), lambda i: (i // bh, i % bh, 0)),
                  pl.BlockSpec((1, _FB, fpad), lambda i: (i // bh, i % bh, 0))],
        out_specs=pl.BlockSpec((_FB, d), lambda i: (i, 0)),
        out_shape=jax.ShapeDtypeStruct((n, d), jnp.float32),
    )(a, ca)


def _sc_mesh():
    return plsc.VectorSubcoreMesh(core_axis_name="c", subcore_axis_name="s")


@functools.cache
def _make_edge_pass(e, n, d):
    nw = 32
    et = e // nw
    nchunk = et // _C
    ng = d // 16

    @functools.partial(
        pl.kernel,
        mesh=_sc_mesh(),
        out_type=(jax.ShapeDtypeStruct((e, d), jnp.float32),
                  jax.ShapeDtypeStruct((nw, 1, 2 * d), jnp.float32)),
        scratch_types=[
            pltpu.VMEM((nchunk, _C), jnp.int32),
            pltpu.VMEM((nchunk, _C), jnp.int32),
            pltpu.VMEM((_C, d), jnp.float32),
            pltpu.VMEM((_C, d), jnp.float32),
            pltpu.VMEM((_C, d), jnp.float32),
            pltpu.VMEM((_C, d), jnp.float32),
            pltpu.VMEM((1, 2 * d), jnp.float32),
            pltpu.SemaphoreType.DMA,
            pltpu.SemaphoreType.DMA,
            pltpu.SemaphoreType.DMA,
            pltpu.SemaphoreType.DMA,
            pltpu.SemaphoreType.DMA,
        ],
    )
    def edge_pass(p_hbm, q_hbm, src3_hbm, dst3_hbm, h_hbm, st_hbm,
                  idxs, idxd, bufp0, bufq0, bufp1, bufq1, acc,
                  sp0, sq0, sp1, sq1, wbs):
        c = lax.axis_index("c")
        s = lax.axis_index("s")
        wid = s * 2 + c
        zero16 = jnp.zeros((16,), jnp.float32)
        for j in range(2 * ng):
            acc[0, pl.ds(j * 16, 16)] = zero16
        pltpu.sync_copy(src3_hbm.at[wid], idxs)
        pltpu.sync_copy(dst3_hbm.at[wid], idxd)
        row_base = wid * nchunk

        def compute(bufp, bufq):
            for k in range(ng):
                col = k * 16

                def rows(r8, ca):
                    s1, s2 = ca
                    for u in range(8):
                        r = r8 * 8 + u
                        hv = jnp.maximum(
                            bufp[r, pl.ds(col, 16)]
                            + bufq[r, pl.ds(col, 16)], 0.0)
                        bufp[r, pl.ds(col, 16)] = hv
                        s1 = s1 + hv
                        s2 = s2 + hv * hv
                    return (s1, s2)

                s1, s2 = lax.fori_loop(0, _C // 8, rows, (zero16, zero16))
                acc[0, pl.ds(col, 16)] += s1
                acc[0, pl.ds(d + col, 16)] += s2

        def body(i2, carry):
            j0 = i2 * 2
            j1 = j0 + 1
            g0p = pltpu.async_copy(p_hbm.at[idxd.at[j0]], bufp0, sp0)
            g0q = pltpu.async_copy(q_hbm.at[idxs.at[j0]], bufq0, sq0)
            g1p = pltpu.async_copy(p_hbm.at[idxd.at[j1]], bufp1, sp1)
            g1q = pltpu.async_copy(q_hbm.at[idxs.at[j1]], bufq1, sq1)
            g0p.wait()
            g0q.wait()
            compute(bufp0, bufq0)
            wb0 = pltpu.async_copy(
                bufp0, h_hbm.at[pl.ds((row_base + j0) * _C, _C)], wbs)
            g1p.wait()
            g1q.wait()
            compute(bufp1, bufq1)
            wb1 = pltpu.async_copy(
                bufp1, h_hbm.at[pl.ds((row_base + j1) * _C, _C)], wbs)
            wb0.wait()
            wb1.wait()
            return carry

        lax.fori_loop(0, nchunk // 2, body, 0)
        if nchunk % 2:
            j = nchunk - 1
            cp = pltpu.async_copy(p_hbm.at[idxd.at[j]], bufp0, sp0)
            cq = pltpu.async_copy(q_hbm.at[idxs.at[j]], bufq0, sq0)
            cp.wait()
            cq.wait()
            compute(bufp0, bufq0)
            pltpu.sync_copy(bufp0, h_hbm.at[pl.ds((row_base + j) * _C, _C)])
        pltpu.sync_copy(acc, st_hbm.at[wid])

    return edge_pass


@functools.cache
def _make_scatter(e, n, f):
    nh = n // 2
    nloc = nh + 8
    et = e // 16
    nchunk = et // _C
    nr = (nh // 16) // 8 * 8
    nz = nr // 3
    rem = nloc - 16 * nr

    @functools.partial(
        pl.kernel,
        mesh=_sc_mesh(),
        out_type=jax.ShapeDtypeStruct((2, nloc, f), jnp.float32),
        scratch_types=[
            pltpu.VMEM((nchunk, _C), jnp.int32),
            pltpu.VMEM((_C, f), jnp.float32),
            pltpu.VMEM((_C, f), jnp.float32),
            pltpu.VMEM((nz, f), jnp.float32),
            pltpu.VMEM_SHARED((nloc, f), jnp.float32),
            pltpu.SemaphoreType.DMA,
            pltpu.SemaphoreType.DMA,
            pltpu.SemaphoreType.DMA,
        ],
    )
    def scatter(h_hbm, dst16_hbm, out_hbm, idx, buf0, buf1, zbuf, shacc,
                sl0, sl1, sadd):
        c = lax.axis_index("c")
        s = lax.axis_index("s")
        lo = c * nh
        zero16 = jnp.zeros((16,), jnp.float32)

        def zrow(r, carry):
            for k in range(f // 16):
                zbuf[r, pl.ds(k * 16, 16)] = zero16
            return carry

        lax.fori_loop(0, nz, zrow, 0)

        for j in range(3):
            pltpu.sync_copy(zbuf, shacc.at[pl.ds(s * nr + j * nz, nz)])

        @pl.when(s == 15)
        def _():
            pltpu.sync_copy(zbuf.at[pl.ds(0, rem)],
                            shacc.at[pl.ds(16 * nr, rem)])

        pltpu.sync_copy(dst16_hbm.at[s], idx)
        row_base = s * nchunk

        def remap_body(j, carry):
            for g in range(_C // 16):
                v = idx[j, pl.ds(g * 16, 16)]
                inb = (v >= lo) & (v < lo + nh)
                idx[j, pl.ds(g * 16, 16)] = jnp.where(inb, v - lo, nh)
            return carry

        lax.fori_loop(0, nchunk, remap_body, 0)
        plsc.subcore_barrier()

        def chunk_body(i2, carry):
            j0 = i2 * 2
            j1 = j0 + 1
            l0 = pltpu.async_copy(
                h_hbm.at[pl.ds((row_base + j0) * _C, _C)], buf0, sl0)
            l1 = pltpu.async_copy(
                h_hbm.at[pl.ds((row_base + j1) * _C, _C)], buf1, sl1)
            l0.wait()
            a0 = pltpu.async_copy(buf0, shacc.at[idx.at[j0]], sadd, add=True)
            l1.wait()
            a1 = pltpu.async_copy(buf1, shacc.at[idx.at[j1]], sadd, add=True)
            a0.wait()
            a1.wait()
            return carry

        lax.fori_loop(0, nchunk // 2, chunk_body, 0)
        plsc.subcore_barrier()
        pltpu.sync_copy(shacc.at[pl.ds(s * nr, nr)],
                        out_hbm.at[c, pl.ds(s * nr, nr)])

        @pl.when(s == 15)
        def _():
            pltpu.sync_copy(shacc.at[pl.ds(16 * nr, rem)],
                            out_hbm.at[c, pl.ds(16 * nr, rem)])

    return scatter


def _bn_affine(g, bt, s1, s2, count):
    mu = s1 / count
    var = s2 / count - mu * mu
    s = g * lax.rsqrt(var + _EPS)
    return s, bt - s * mu


def kernel(x, edge_index, bn0, enc, dec):
    n, d = x.shape
    e = edge_index.shape[1]
    src = edge_index[0]
    dst = edge_index[1]
    src3 = src.reshape(32, e // (32 * _C), _C)
    dst3 = dst.reshape(32, e // (32 * _C), _C)
    dst16 = dst.reshape(16, e // (16 * _C), _C)

    s1x, s2x = _colstats(x)
    s0, t0 = _bn_affine(bn0[0], bn0[1], s1x, s2x, n)

    W1, b1 = enc[0][0], enc[0][1]
    W1a, W1b = W1[:, :d], W1[:, d:]
    A1 = (W1a - W1b) * s0[None, :]
    B1 = W1b * s0[None, :]
    b1p = b1 + W1a @ t0
    big = W1.shape[0]
    wt1 = jnp.concatenate([A1, B1], axis=0).T
    bias1 = jnp.concatenate([b1p, jnp.zeros_like(b1p)])[None, :]
    p1, q1 = _pq(x, wt1, bias1)

    edge_pass = _make_edge_pass(e, n, big)
    h, st = edge_pass(p1, q1, src3, dst3)
    stsum = jnp.sum(st[:, 0, :], axis=0)
    s, t = _bn_affine(enc[0][2], enc[0][3], stsum[:big], stsum[big:], e)

    for layer in enc[1:-1]:
        W, b, g, bt = layer
        wt = (W * s[None, :]).T
        bp = (b + W @ t)[None, :]
        h, hs1, hs2 = _mm_stats(h, wt, bp, relu=True)
        s, t = _bn_affine(g, bt, hs1, hs2, e)

    W, b, g, bt = enc[-1]
    hid = W.shape[0]
    fpad = 128
    wt = jnp.zeros((big, fpad), jnp.float32).at[:, :hid].set((W * s[None, :]).T)
    bp = jnp.zeros((1, fpad), jnp.float32).at[:, :hid].set((b + W @ t)[None, :])
    bp = bp.at[0, hid].set(1.0)
    h, hs1, hs2 = _mm_stats(h, wt, bp, relu=True)
    s, t = _bn_affine(g, bt, hs1[:hid], hs2[:hid], e)

    scat = _make_scatter(e, n, fpad)
    sums = scat(h, dst16)

    V1, c1 = dec[0][0], dec[0][1]
    V1a, V1b = V1[:, :hid], V1[:, hid:]
    wt2 = jnp.concatenate([V1a - V1b, V1b], axis=0).T
    bias2 = jnp.concatenate([c1, jnp.zeros_like(c1)])[None, :]
    p2, q2 = _fin_pq(n, hid, sums, s[None, :], t[None, :], wt2, bias2)

    h, st = edge_pass(p2, q2, src3, dst3)
    stsum = jnp.sum(st[:, 0, :], axis=0)
    s, t = _bn_affine(dec[0][2], dec[0][3], stsum[:big], stsum[big:], e)

    for layer in dec[1:-1]:
        W, b, g, bt = layer
        wt = (W * s[None, :]).T
        bp = (b + W @ t)[None, :]
        h, hs1, hs2 = _mm_stats(h, wt, bp, relu=True)
        s, t = _bn_affine(g, bt, hs1, hs2, e)

    Wf, bf = dec[-1]
    wtf = (Wf * s[None, :]).T
    bfp = (bf + Wf @ t)[None, :]
    h = _mm_plain(h, wtf, bfp)

    sums2 = scat(h, dst16)
    return _fin_out(n, hid, sums2, sums)


# speedup vs baseline: 1.9829x; 1.0121x over previous
import functools

import jax
import jax.numpy as jnp
from jax import lax
from jax.experimental import pallas as pl
from jax.experimental.pallas import tpu as pltpu
from jax.experimental.pallas import tpu_sc as plsc

_EPS = 1e-5
_BN = 2000
_BE = 3200
_C = 80


def _colstats_body(x_ref, s1_ref, s2_ref):
    i = pl.program_id(0)
    x = x_ref[...]
    d = x.shape[-1]

    @pl.when(i == 0)
    def _():
        s1_ref[...] = jnp.zeros_like(s1_ref)
        s2_ref[...] = jnp.zeros_like(s2_ref)

    s1_ref[...] += jnp.sum(x.reshape(-1, 8, d), axis=0)
    s2_ref[...] += jnp.sum((x * x).reshape(-1, 8, d), axis=0)


def _colstats(x):
    n, d = x.shape
    s1, s2 = pl.pallas_call(
        _colstats_body,
        grid=(n // _BN,),
        in_specs=[pl.BlockSpec((_BN, d), lambda i: (i, 0))],
        out_specs=(pl.BlockSpec((8, d), lambda i: (0, 0)),
                   pl.BlockSpec((8, d), lambda i: (0, 0))),
        out_shape=(jax.ShapeDtypeStruct((8, d), jnp.float32),
                   jax.ShapeDtypeStruct((8, d), jnp.float32)),
    )(x)
    return jnp.sum(s1, axis=0), jnp.sum(s2, axis=0)


def _mm_stats_body(relu, out_bf16, x_ref, wt_ref, b_ref,
                   h_ref, s1_ref, s2_ref):
    i = pl.program_id(0)
    z = jnp.dot(x_ref[...], wt_ref[...],
                preferred_element_type=jnp.float32,
                precision=lax.Precision.HIGHEST) + b_ref[...]
    if relu:
        z = jnp.maximum(z, 0.0)
    if out_bf16:
        hc = z.astype(jnp.bfloat16)
        h_ref[...] = hc
        z = hc.astype(jnp.float32)
    else:
        h_ref[...] = z
    d = z.shape[-1]

    @pl.when(i == 0)
    def _():
        s1_ref[...] = jnp.zeros_like(s1_ref)
        s2_ref[...] = jnp.zeros_like(s2_ref)

    s1_ref[...] += jnp.sum(z.reshape(-1, 8, d), axis=0)
    s2_ref[...] += jnp.sum((z * z).reshape(-1, 8, d), axis=0)


def _mm_stats(x, wt, b, relu=True, out_bf16=False):
    e, din = x.shape
    dout = wt.shape[1]
    odt = jnp.bfloat16 if out_bf16 else jnp.float32
    h, s1, s2 = pl.pallas_call(
        functools.partial(_mm_stats_body, relu, out_bf16),
        grid=(e // _BE,),
        in_specs=[pl.BlockSpec((_BE, din), lambda i: (i, 0)),
                  pl.BlockSpec((din, dout), lambda i: (0, 0)),
                  pl.BlockSpec((1, dout), lambda i: (0, 0))],
        out_specs=(pl.BlockSpec((_BE, dout), lambda i: (i, 0)),
                   pl.BlockSpec((8, dout), lambda i: (0, 0)),
                   pl.BlockSpec((8, dout), lambda i: (0, 0))),
        out_shape=(jax.ShapeDtypeStruct((e, dout), odt),
                   jax.ShapeDtypeStruct((8, dout), jnp.float32),
                   jax.ShapeDtypeStruct((8, dout), jnp.float32)),
    )(x, wt, b)
    return h, jnp.sum(s1, axis=0), jnp.sum(s2, axis=0)


def _mm_body(x_ref, wt_ref, b_ref, h_ref):
    xb = x_ref[...].astype(jnp.bfloat16)
    h_ref[...] = jnp.dot(xb, wt_ref[...].astype(jnp.bfloat16),
                         preferred_element_type=jnp.float32) + b_ref[...]


def _mm_plain(x, wt, b):
    e, din = x.shape
    dout = wt.shape[1]
    return pl.pallas_call(
        _mm_body,
        grid=(e // _BE,),
        in_specs=[pl.BlockSpec((_BE, din), lambda i: (i, 0)),
                  pl.BlockSpec((din, dout), lambda i: (0, 0)),
                  pl.BlockSpec((1, dout), lambda i: (0, 0))],
        out_specs=pl.BlockSpec((_BE, dout), lambda i: (i, 0)),
        out_shape=jax.ShapeDtypeStruct((e, dout), jnp.float32),
    )(x, wt, b)


def _pq_body(x_ref, wt_ref, b_ref, p_ref, q_ref):
    z = jnp.dot(x_ref[...], wt_ref[...],
                preferred_element_type=jnp.float32,
                precision=lax.Precision.HIGHEST) + b_ref[...]
    h = z.shape[-1] // 2
    p_ref[...] = z[:, :h]
    q_ref[...] = z[:, h:]


def _pq(x, wt, b):
    n, din = x.shape
    dout = wt.shape[1] // 2
    return pl.pallas_call(
        _pq_body,
        grid=(n // _BN,),
        in_specs=[pl.BlockSpec((_BN, din), lambda i: (i, 0)),
                  pl.BlockSpec((din, 2 * dout), lambda i: (0, 0)),
                  pl.BlockSpec((1, 2 * dout), lambda i: (0, 0))],
        out_specs=(pl.BlockSpec((_BN, dout), lambda i: (i, 0)),
                   pl.BlockSpec((_BN, dout), lambda i: (i, 0))),
        out_shape=(jax.ShapeDtypeStruct((n, dout), jnp.float32),
                   jax.ShapeDtypeStruct((n, dout), jnp.float32)),
    )(x, wt, b)


_FB = 1000


def _fin_pq_body(hid, a_ref, s_ref, t_ref, wt_ref, b_ref, p_ref, q_ref):
    a = a_ref[0]
    cnt = a[:, hid:hid + 1]
    mean = a[:, :hid] / jnp.maximum(cnt, 1.0)
    g = jnp.where(cnt > 0, mean * s_ref[...] + t_ref[...], 0.0)
    z = jnp.dot(g, wt_ref[...], preferred_element_type=jnp.float32,
                precision=lax.Precision.HIGHEST) + b_ref[...]
    h = z.shape[-1] // 2
    p_ref[...] = z[:, :h]
    q_ref[...] = z[:, h:]


def _fin_pq(n, hid, a, s, t, wt, b):
    fpad = a.shape[-1]
    dout = wt.shape[1] // 2
    bh = (n // 2) // _FB
    return pl.pallas_call(
        functools.partial(_fin_pq_body, hid),
        grid=(n // _FB,),
        in_specs=[pl.BlockSpec((1, _FB, fpad), lambda i: (i // bh, i % bh, 0)),
                  pl.BlockSpec((1, hid), lambda i: (0, 0)),
                  pl.BlockSpec((1, hid), lambda i: (0, 0)),
                  pl.BlockSpec((hid, 2 * dout), lambda i: (0, 0)),
                  pl.BlockSpec((1, 2 * dout), lambda i: (0, 0))],
        out_specs=(pl.BlockSpec((_FB, dout), lambda i: (i, 0)),
                   pl.BlockSpec((_FB, dout), lambda i: (i, 0))),
        out_shape=(jax.ShapeDtypeStruct((n, dout), jnp.float32),
                   jax.ShapeDtypeStruct((n, dout), jnp.float32)),
    )(a, s, t, wt, b)


def _fin_out_body(hid, a_ref, ca_ref, o_ref):
    cnt = ca_ref[0][:, hid:hid + 1]
    m = a_ref[0] / jnp.maximum(cnt, 1.0)
    o_ref[...] = jnp.where(cnt > 0, m, 0.0)


def _fin_out(n, hid, a, ca):
    d = a.shape[-1]
    fpad = ca.shape[-1]
    bh = (n // 2) // _FB
    return pl.pallas_call(
        functools.partial(_fin_out_body, hid),
        grid=(n // _FB,),
        in_specs=[pl.BlockSpec((1, _FB, d), lambda i: (i // bh, i % bh, 0)),
                  pl.BlockSpec((1, _FB, fpad), lambda i: (i // bh, i % bh, 0))],
        out_specs=pl.BlockSpec((_FB, d), lambda i: (i, 0)),
        out_shape=jax.ShapeDtypeStruct((n, d), jnp.float32),
    )(a, ca)


def _sc_mesh():
    return plsc.VectorSubcoreMesh(core_axis_name="c", subcore_axis_name="s")


@functools.cache
def _make_edge_pass(e, n, d):
    nw = 32
    et = e // nw
    nchunk = et // _C
    ng = d // 16

    @functools.partial(
        pl.kernel,
        mesh=_sc_mesh(),
        out_type=(jax.ShapeDtypeStruct((e, d), jnp.float32),
                  jax.ShapeDtypeStruct((nw, 1, 2 * d), jnp.float32)),
        scratch_types=[
            pltpu.VMEM((nchunk, _C), jnp.int32),
            pltpu.VMEM((nchunk, _C), jnp.int32),
            pltpu.VMEM((_C, d), jnp.float32),
            pltpu.VMEM((_C, d), jnp.float32),
            pltpu.VMEM((_C, d), jnp.float32),
            pltpu.VMEM((_C, d), jnp.float32),
            pltpu.VMEM((1, 2 * d), jnp.float32),
            pltpu.SemaphoreType.DMA,
            pltpu.SemaphoreType.DMA,
            pltpu.SemaphoreType.DMA,
            pltpu.SemaphoreType.DMA,
            pltpu.SemaphoreType.DMA,
        ],
    )
    def edge_pass(p_hbm, q_hbm, src3_hbm, dst3_hbm, h_hbm, st_hbm,
                  idxs, idxd, bufp0, bufq0, bufp1, bufq1, acc,
                  sp0, sq0, sp1, sq1, wbs):
        c = lax.axis_index("c")
        s = lax.axis_index("s")
        wid = s * 2 + c
        zero16 = jnp.zeros((16,), jnp.float32)
        for j in range(2 * ng):
            acc[0, pl.ds(j * 16, 16)] = zero16
        pltpu.sync_copy(src3_hbm.at[wid], idxs)
        pltpu.sync_copy(dst3_hbm.at[wid], idxd)
        row_base = wid * nchunk

        def compute(bufp, bufq):
            for k in range(ng):
                col = k * 16

                def rows(r8, ca):
                    s1, s2 = ca
                    for u in range(8):
                        r = r8 * 8 + u
                        hv = jnp.maximum(
                            bufp[r, pl.ds(col, 16)]
                            + bufq[r, pl.ds(col, 16)], 0.0)
                        bufp[r, pl.ds(col, 16)] = hv
                        s1 = s1 + hv
                        s2 = s2 + hv * hv
                    return (s1, s2)

                s1, s2 = lax.fori_loop(0, _C // 8, rows, (zero16, zero16))
                acc[0, pl.ds(col, 16)] += s1
                acc[0, pl.ds(d + col, 16)] += s2

        def body(i2, carry):
            j0 = i2 * 2
            j1 = j0 + 1
            g0p = pltpu.async_copy(p_hbm.at[idxd.at[j0]], bufp0, sp0)
            g0q = pltpu.async_copy(q_hbm.at[idxs.at[j0]], bufq0, sq0)
            g1p = pltpu.async_copy(p_hbm.at[idxd.at[j1]], bufp1, sp1)
            g1q = pltpu.async_copy(q_hbm.at[idxs.at[j1]], bufq1, sq1)
            g0p.wait()
            g0q.wait()
            compute(bufp0, bufq0)
            wb0 = pltpu.async_copy(
                bufp0, h_hbm.at[pl.ds((row_base + j0) * _C, _C)], wbs)
            g1p.wait()
            g1q.wait()
            compute(bufp1, bufq1)
            wb1 = pltpu.async_copy(
                bufp1, h_hbm.at[pl.ds((row_base + j1) * _C, _C)], wbs)
            wb0.wait()
            wb1.wait()
            return carry

        lax.fori_loop(0, nchunk // 2, body, 0)
        if nchunk % 2:
            j = nchunk - 1
            cp = pltpu.async_copy(p_hbm.at[idxd.at[j]], bufp0, sp0)
            cq = pltpu.async_copy(q_hbm.at[idxs.at[j]], bufq0, sq0)
            cp.wait()
            cq.wait()
            compute(bufp0, bufq0)
            pltpu.sync_copy(bufp0, h_hbm.at[pl.ds((row_base + j) * _C, _C)])
        pltpu.sync_copy(acc, st_hbm.at[wid])

    return edge_pass


@functools.cache
def _make_scatter(e, n, f):
    nh = n // 2
    nloc = nh + 8
    et = e // 16
    nchunk = et // _C
    nr = (nh // 16) // 8 * 8
    nz = nr // 3
    rem = nloc - 16 * nr

    @functools.partial(
        pl.kernel,
        mesh=_sc_mesh(),
        out_type=jax.ShapeDtypeStruct((2, nloc, f), jnp.float32),
        scratch_types=[
            pltpu.VMEM((nchunk, _C), jnp.int32),
            pltpu.VMEM((_C, f), jnp.float32),
            pltpu.VMEM((_C, f), jnp.float32),
            pltpu.VMEM((_C, f), jnp.float32),
            pltpu.VMEM((_C, f), jnp.float32),
            pltpu.VMEM((nz, f), jnp.float32),
            pltpu.VMEM_SHARED((nloc, f), jnp.float32),
            pltpu.SemaphoreType.DMA,
            pltpu.SemaphoreType.DMA,
            pltpu.SemaphoreType.DMA,
            pltpu.SemaphoreType.DMA,
            pltpu.SemaphoreType.DMA,
        ],
    )
    def scatter(h_hbm, dst16_hbm, out_hbm, idx, buf0, buf1, buf2, buf3,
                zbuf, shacc, sl0, sl1, sl2, sl3, sadd):
        c = lax.axis_index("c")
        s = lax.axis_index("s")
        lo = c * nh
        zero16 = jnp.zeros((16,), jnp.float32)

        def zrow(r, carry):
            for k in range(f // 16):
                zbuf[r, pl.ds(k * 16, 16)] = zero16
            return carry

        lax.fori_loop(0, nz, zrow, 0)

        for j in range(3):
            pltpu.sync_copy(zbuf, shacc.at[pl.ds(s * nr + j * nz, nz)])

        @pl.when(s == 15)
        def _():
            pltpu.sync_copy(zbuf.at[pl.ds(0, rem)],
                            shacc.at[pl.ds(16 * nr, rem)])

        pltpu.sync_copy(dst16_hbm.at[s], idx)
        row_base = s * nchunk

        def remap_body(j, carry):
            for g in range(_C // 16):
                v = idx[j, pl.ds(g * 16, 16)]
                inb = (v >= lo) & (v < lo + nh)
                idx[j, pl.ds(g * 16, 16)] = jnp.where(inb, v - lo, nh)
            return carry

        lax.fori_loop(0, nchunk, remap_body, 0)
        plsc.subcore_barrier()

        bufs = (buf0, buf1, buf2, buf3)
        sls = (sl0, sl1, sl2, sl3)

        def chunk_body(i4, carry):
            j0 = i4 * 4
            loads = [
                pltpu.async_copy(
                    h_hbm.at[pl.ds((row_base + j0 + u) * _C, _C)],
                    bufs[u], sls[u])
                for u in range(4)
            ]
            adds = []
            for u in range(4):
                loads[u].wait()
                adds.append(pltpu.async_copy(
                    bufs[u], shacc.at[idx.at[j0 + u]], sadd, add=True))
            for a in adds:
                a.wait()
            return carry

        lax.fori_loop(0, nchunk // 4, chunk_body, 0)
        for j in range(nchunk // 4 * 4, nchunk):
            pltpu.sync_copy(h_hbm.at[pl.ds((row_base + j) * _C, _C)], buf0)
            pltpu.sync_copy(buf0, shacc.at[idx.at[j]], add=True)
        plsc.subcore_barrier()
        pltpu.sync_copy(shacc.at[pl.ds(s * nr, nr)],
                        out_hbm.at[c, pl.ds(s * nr, nr)])

        @pl.when(s == 15)
        def _():
            pltpu.sync_copy(shacc.at[pl.ds(16 * nr, rem)],
                            out_hbm.at[c, pl.ds(16 * nr, rem)])

    return scatter


def _bn_affine(g, bt, s1, s2, count):
    mu = s1 / count
    var = s2 / count - mu * mu
    s = g * lax.rsqrt(var + _EPS)
    return s, bt - s * mu


def kernel(x, edge_index, bn0, enc, dec):
    n, d = x.shape
    e = edge_index.shape[1]
    src = edge_index[0]
    dst = edge_index[1]
    src3 = src.reshape(32, e // (32 * _C), _C)
    dst3 = dst.reshape(32, e // (32 * _C), _C)
    dst16 = dst.reshape(16, e // (16 * _C), _C)

    s1x, s2x = _colstats(x)
    s0, t0 = _bn_affine(bn0[0], bn0[1], s1x, s2x, n)

    W1, b1 = enc[0][0], enc[0][1]
    W1a, W1b = W1[:, :d], W1[:, d:]
    A1 = (W1a - W1b) * s0[None, :]
    B1 = W1b * s0[None, :]
    b1p = b1 + W1a @ t0
    big = W1.shape[0]
    wt1 = jnp.concatenate([A1, B1], axis=0).T
    bias1 = jnp.concatenate([b1p, jnp.zeros_like(b1p)])[None, :]
    p1, q1 = _pq(x, wt1, bias1)

    edge_pass = _make_edge_pass(e, n, big)
    h, st = edge_pass(p1, q1, src3, dst3)
    stsum = jnp.sum(st[:, 0, :], axis=0)
    s, t = _bn_affine(enc[0][2], enc[0][3], stsum[:big], stsum[big:], e)

    for layer in enc[1:-1]:
        W, b, g, bt = layer
        wt = (W * s[None, :]).T
        bp = (b + W @ t)[None, :]
        h, hs1, hs2 = _mm_stats(h, wt, bp, relu=True)
        s, t = _bn_affine(g, bt, hs1, hs2, e)

    W, b, g, bt = enc[-1]
    hid = W.shape[0]
    fpad = 128
    wt = jnp.zeros((big, fpad), jnp.float32).at[:, :hid].set((W * s[None, :]).T)
    bp = jnp.zeros((1, fpad), jnp.float32).at[:, :hid].set((b + W @ t)[None, :])
    bp = bp.at[0, hid].set(1.0)
    h, hs1, hs2 = _mm_stats(h, wt, bp, relu=True)
    s, t = _bn_affine(g, bt, hs1[:hid], hs2[:hid], e)

    scat = _make_scatter(e, n, fpad)
    sums = scat(h, dst16)

    V1, c1 = dec[0][0], dec[0][1]
    V1a, V1b = V1[:, :hid], V1[:, hid:]
    wt2 = jnp.concatenate([V1a - V1b, V1b], axis=0).T
    bias2 = jnp.concatenate([c1, jnp.zeros_like(c1)])[None, :]
    p2, q2 = _fin_pq(n, hid, sums, s[None, :], t[None, :], wt2, bias2)

    h, st = edge_pass(p2, q2, src3, dst3)
    stsum = jnp.sum(st[:, 0, :], axis=0)
    s, t = _bn_affine(dec[0][2], dec[0][3], stsum[:big], stsum[big:], e)

    for layer in dec[1:-1]:
        W, b, g, bt = layer
        wt = (W * s[None, :]).T
        bp = (b + W @ t)[None, :]
        h, hs1, hs2 = _mm_stats(h, wt, bp, relu=True)
        s, t = _bn_affine(g, bt, hs1, hs2, e)

    Wf, bf = dec[-1]
    wtf = (Wf * s[None, :]).T
    bfp = (bf + Wf @ t)[None, :]
    h = _mm_plain(h, wtf, bfp)

    sums2 = scat(h, dst16)
    return _fin_out(n, hid, sums2, sums)
